# baseline (speedup 1.0000x reference)
import functools

import jax
import jax.numpy as jnp
from jax import lax
from jax.experimental import pallas as pl
from jax.experimental.pallas import tpu as pltpu
from jax.experimental.pallas import tpu_sc as plsc

B = 4096
L = 200
D = 64
NB = B * L
N_TOK = 1000000
SCALE = 8.0

_INFO = plsc.get_sparse_core_info()
NC = _INFO.num_cores
NS = _INFO.num_subcores
NW = NC * NS
BPW = NB // NW
C = 256
NCHUNK = BPW // C

_mesh = plsc.VectorSubcoreMesh(core_axis_name="c", subcore_axis_name="s")


@functools.partial(
    pl.kernel,
    mesh=_mesh,
    compiler_params=pltpu.CompilerParams(use_tc_tiling_on_sc=True),
    out_type=jax.ShapeDtypeStruct((NB, D), jnp.float32),
    scratch_types=[
        pltpu.VMEM((C,), jnp.int32),
        pltpu.VMEM((C, 2 * D), jnp.float32),
        pltpu.VMEM((C, D), jnp.float32),
        pltpu.SemaphoreType.DMA,
    ],
)
def _emb(idx_hbm, tw_hbm, out_hbm, idx_v, wide_v, out_v, sem):
    wid = lax.axis_index("s") * NC + lax.axis_index("c")
    base = wid * BPW

    def chunk(g, carry):
        off = base + g * C
        pltpu.sync_copy(idx_hbm.at[pl.ds(off, C)], idx_v)
        pltpu.async_copy(tw_hbm.at[idx_v], wide_v, sem).wait()

        def row(t, c):
            for j in range(D // 16):
                sl = pl.ds(j * 16, 16)
                out_v[t, sl] = wide_v[t, sl] * SCALE
            return c

        lax.fori_loop(0, C, row, 0, unroll=2)
        pltpu.sync_copy(out_v, out_hbm.at[pl.ds(off, C)])
        return carry

    lax.fori_loop(0, NCHUNK, chunk, 0)


def kernel(x, table):
    idx = x.reshape(NB).astype(jnp.int32)
    tw = jnp.pad(table, ((0, 0), (0, D)))
    out = _emb(idx, tw)
    return out.reshape(B, L, D)


# speedup vs baseline: 1.1507x; 1.1507x over previous
import functools

import jax
import jax.numpy as jnp
from jax import lax
from jax.experimental import pallas as pl
from jax.experimental.pallas import tpu as pltpu
from jax.experimental.pallas import tpu_sc as plsc

B = 4096
L = 200
D = 64
NB = B * L
N_TOK = 1000000
SCALE = 8.0

_INFO = plsc.get_sparse_core_info()
NC = _INFO.num_cores
NS = _INFO.num_subcores
NW = NC * NS
BPW = NB // NW
C = 256
NCHUNK = BPW // C

_mesh = plsc.VectorSubcoreMesh(core_axis_name="c", subcore_axis_name="s")


@functools.partial(
    pl.kernel,
    mesh=_mesh,
    compiler_params=pltpu.CompilerParams(use_tc_tiling_on_sc=True),
    out_type=jax.ShapeDtypeStruct((NB, D), jnp.float32),
    scratch_types=[
        pltpu.VMEM((C,), jnp.int32),
        pltpu.VMEM((C,), jnp.int32),
        pltpu.VMEM((C, 2 * D), jnp.float32),
        pltpu.VMEM((C, 2 * D), jnp.float32),
        pltpu.VMEM((C, D), jnp.float32),
        pltpu.SemaphoreType.DMA,
        pltpu.SemaphoreType.DMA,
    ],
)
def _emb(idx_hbm, tw_hbm, out_hbm,
         idx0, idx1, wide0, wide1, st, sem0, sem1):
    wid = lax.axis_index("s") * NC + lax.axis_index("c")
    base = wid * BPW
    idx_v = (idx0, idx1)
    wide_v = (wide0, wide1)
    sems = (sem0, sem1)

    def issue(g, b):
        off = base + g * C
        pltpu.sync_copy(idx_hbm.at[pl.ds(off, C)], idx_v[b])
        pltpu.async_copy(tw_hbm.at[idx_v[b]], wide_v[b], sems[b])

    def drain_and_flush(g, b):
        pltpu.make_async_copy(tw_hbm.at[idx_v[b]], wide_v[b], sems[b]).wait()

        def row(t, c):
            for j in range(D // 16):
                sl = pl.ds(j * 16, 16)
                st[t, sl] = wide_v[b][t, sl] * SCALE
            return c

        lax.fori_loop(0, C, row, 0, unroll=4)
        pltpu.sync_copy(st, out_hbm.at[pl.ds(base + g * C, C)])

        @pl.when(g + 2 < NCHUNK)
        def _():
            issue(g + 2, b)

    issue(0, 0)
    issue(1, 1)

    def pair(i, carry):
        g = i * 2
        drain_and_flush(g, 0)
        drain_and_flush(g + 1, 1)
        return carry

    lax.fori_loop(0, NCHUNK // 2, pair, 0)


def kernel(x, table):
    idx = x.reshape(NB).astype(jnp.int32)
    tw = jnp.pad(table, ((0, 0), (0, D)))
    out = _emb(idx, tw)
    return out.reshape(B, L, D)
